# baseline (speedup 1.0000x reference)
import jax
import jax.numpy as jnp
from jax import lax
from jax.experimental import pallas as pl
from jax.experimental.pallas import tpu as pltpu

_B, _L = 16384, 50
_GRID = 2
_NSPLIT = 2
_BC = _B // (_GRID * _NSPLIT)


def _body(*refs):
    out_ref, acc_ref = refs[6 * _NSPLIT], refs[6 * _NSPLIT + 1]
    b = pl.program_id(0)
    shape = (_L, _BC)
    row = lax.broadcasted_iota(jnp.int32, shape, 0)
    zero = jnp.zeros(shape, jnp.float32)
    one = jnp.ones(shape, jnp.float32)

    for j in range(_NSPLIT):
        seq_ref, fg_ref, slp_ref, bnl_ref, fgl_ref, r_ref = (
            refs[0 * _NSPLIT + j], refs[1 * _NSPLIT + j],
            refs[2 * _NSPLIT + j], refs[3 * _NSPLIT + j],
            refs[4 * _NSPLIT + j], refs[5 * _NSPLIT + j])
        prev = pltpu.roll(seq_ref[...], 1, 0)
        smask = jnp.logical_or(row == 0, prev > 0)
        r = r_ref[...]
        x = slp_ref[...] * r
        bmask = fg_ref[...] > 0
        p = (jnp.sum(jnp.where(smask, x, zero), axis=0),
             jnp.sum(jnp.where(smask, one, zero), axis=0),
             jnp.sum(jnp.where(bmask, bnl_ref[...] * r, zero), axis=0),
             jnp.sum(jnp.where(bmask, fgl_ref[...] * r, zero), axis=0),
             jnp.sum(jnp.where(bmask, one, zero), axis=0))

        if j == 0:
            @pl.when(b == 0)
            def _():
                for i in range(5):
                    acc_ref[i] = p[i]

            @pl.when(b > 0)
            def _():
                for i in range(5):
                    acc_ref[i] += p[i]
        else:
            for i in range(5):
                acc_ref[i] += p[i]

    @pl.when(b == _GRID - 1)
    def _():
        s1 = jnp.sum(acc_ref[0])
        c1 = jnp.sum(acc_ref[1])
        s2 = jnp.sum(acc_ref[2])
        s3 = jnp.sum(acc_ref[3])
        c2 = jnp.maximum(jnp.sum(acc_ref[4]), 1.0)
        lane = lax.broadcasted_iota(jnp.int32, (1, 128), 1)
        out_ref[...] = jnp.where(
            lane == 0, -s1 / c1, jnp.where(lane == 1, -s2 / c2, -s3 / c2))


def _spec(j):
    return pl.BlockSpec((_L, _BC), lambda b, jj=j: (0, _NSPLIT * b + jj))


_call = pl.pallas_call(
    _body,
    grid=(_GRID,),
    in_specs=[_spec(j) for _ in range(6) for j in range(_NSPLIT)],
    out_specs=pl.BlockSpec((1, 128), lambda b: (0, 0)),
    out_shape=jax.ShapeDtypeStruct((1, 128), jnp.float32),
    scratch_shapes=[pltpu.VMEM((5, _BC), jnp.float32)],
    compiler_params=pltpu.CompilerParams(
        dimension_semantics=("arbitrary",),
    ),
)


def kernel(seq, bn_seq, fg_seq, seqLogprobs, bnLogprobs, fgLogprobs, reward):
    del bn_seq
    arrs = (seq.T.astype(jnp.int32), fg_seq.T.astype(jnp.int32),
            seqLogprobs.T, bnLogprobs.T, fgLogprobs.T, reward.T)
    out = _call(*[a for a in arrs for _ in range(_NSPLIT)])
    return (out[0, 0], out[0, 1], out[0, 2])


# speedup vs baseline: 1.0325x; 1.0325x over previous
import jax
import jax.numpy as jnp
from jax import lax
from jax.experimental import pallas as pl
from jax.experimental.pallas import tpu as pltpu

_B, _L = 16384, 50
_GRID = 4
_NSPLIT = 2
_BC = _B // (_GRID * _NSPLIT)


def _body(*refs):
    out_ref, acc_ref = refs[6 * _NSPLIT], refs[6 * _NSPLIT + 1]
    b = pl.program_id(0)
    shape = (_L, _BC)
    row = lax.broadcasted_iota(jnp.int32, shape, 0)
    zero = jnp.zeros(shape, jnp.float32)
    one = jnp.ones(shape, jnp.float32)

    for j in range(_NSPLIT):
        seq_ref, fg_ref, slp_ref, bnl_ref, fgl_ref, r_ref = (
            refs[0 * _NSPLIT + j], refs[1 * _NSPLIT + j],
            refs[2 * _NSPLIT + j], refs[3 * _NSPLIT + j],
            refs[4 * _NSPLIT + j], refs[5 * _NSPLIT + j])
        prev = pltpu.roll(seq_ref[...], 1, 0)
        smask = jnp.logical_or(row == 0, prev > 0)
        r = r_ref[...]
        x = slp_ref[...] * r
        bmask = fg_ref[...] > 0
        p = (jnp.sum(jnp.where(smask, x, zero), axis=0),
             jnp.sum(jnp.where(smask, one, zero), axis=0),
             jnp.sum(jnp.where(bmask, bnl_ref[...] * r, zero), axis=0),
             jnp.sum(jnp.where(bmask, fgl_ref[...] * r, zero), axis=0),
             jnp.sum(jnp.where(bmask, one, zero), axis=0))

        if j == 0:
            @pl.when(b == 0)
            def _():
                for i in range(5):
                    acc_ref[i] = p[i]

            @pl.when(b > 0)
            def _():
                for i in range(5):
                    acc_ref[i] += p[i]
        else:
            for i in range(5):
                acc_ref[i] += p[i]

    @pl.when(b == _GRID - 1)
    def _():
        s1 = jnp.sum(acc_ref[0])
        c1 = jnp.sum(acc_ref[1])
        s2 = jnp.sum(acc_ref[2])
        s3 = jnp.sum(acc_ref[3])
        c2 = jnp.maximum(jnp.sum(acc_ref[4]), 1.0)
        lane = lax.broadcasted_iota(jnp.int32, (1, 128), 1)
        out_ref[...] = jnp.where(
            lane == 0, -s1 / c1, jnp.where(lane == 1, -s2 / c2, -s3 / c2))


def _spec(j):
    return pl.BlockSpec((_L, _BC), lambda b, jj=j: (0, _NSPLIT * b + jj))


_call = pl.pallas_call(
    _body,
    grid=(_GRID,),
    in_specs=[_spec(j) for _ in range(6) for j in range(_NSPLIT)],
    out_specs=pl.BlockSpec((1, 128), lambda b: (0, 0)),
    out_shape=jax.ShapeDtypeStruct((1, 128), jnp.float32),
    scratch_shapes=[pltpu.VMEM((5, _BC), jnp.float32)],
    compiler_params=pltpu.CompilerParams(
        dimension_semantics=("arbitrary",),
    ),
)


def kernel(seq, bn_seq, fg_seq, seqLogprobs, bnLogprobs, fgLogprobs, reward):
    del bn_seq
    arrs = (seq.T.astype(jnp.int32), fg_seq.T.astype(jnp.int32),
            seqLogprobs.T, bnLogprobs.T, fgLogprobs.T, reward.T)
    out = _call(*[a for a in arrs for _ in range(_NSPLIT)])
    return (out[0, 0], out[0, 1], out[0, 2])


# speedup vs baseline: 1.0325x; 1.0000x over previous
import jax
import jax.numpy as jnp
from jax import lax
from jax.experimental import pallas as pl
from jax.experimental.pallas import tpu as pltpu

_B, _L = 16384, 50
_GRID = 4
_NSPLIT = 2
_BC = _B // (_GRID * _NSPLIT)


def _body(*refs):
    out_ref, acc_ref = refs[6 * _NSPLIT], refs[6 * _NSPLIT + 1]
    b = pl.program_id(0)
    shape = (_L, _BC)
    row = lax.broadcasted_iota(jnp.int32, shape, 0)
    zero = jnp.zeros(shape, jnp.float32)
    one = jnp.ones(shape, jnp.float32)

    for j in range(_NSPLIT):
        seq_ref, fg_ref, slp_ref, bnl_ref, fgl_ref, r_ref = (
            refs[0 * _NSPLIT + j], refs[1 * _NSPLIT + j],
            refs[2 * _NSPLIT + j], refs[3 * _NSPLIT + j],
            refs[4 * _NSPLIT + j], refs[5 * _NSPLIT + j])
        prev = pltpu.roll(seq_ref[...], 1, 0)
        smask = jnp.logical_or(row == 0, prev > 0)
        r = r_ref[...]
        x = slp_ref[...] * r
        bmask = fg_ref[...] > 0
        p = (jnp.sum(jnp.where(smask, x, zero), axis=0),
             jnp.sum(jnp.where(smask, one, zero), axis=0),
             jnp.sum(jnp.where(bmask, bnl_ref[...] * r, zero), axis=0),
             jnp.sum(jnp.where(bmask, fgl_ref[...] * r, zero), axis=0),
             jnp.sum(jnp.where(bmask, one, zero), axis=0))

        if j == 0:
            @pl.when(b == 0)
            def _():
                for i in range(5):
                    acc_ref[i] = p[i]

            @pl.when(b > 0)
            def _():
                for i in range(5):
                    acc_ref[i] += p[i]
        else:
            for i in range(5):
                acc_ref[i] += p[i]

    @pl.when(b == _GRID - 1)
    def _():
        s1 = jnp.sum(acc_ref[0])
        c1 = jnp.sum(acc_ref[1])
        s2 = jnp.sum(acc_ref[2])
        s3 = jnp.sum(acc_ref[3])
        c2 = jnp.maximum(jnp.sum(acc_ref[4]), 1.0)
        lane = lax.broadcasted_iota(jnp.int32, (1, 128), 1)
        out_ref[...] = jnp.where(
            lane == 0, -s1 / c1, jnp.where(lane == 1, -s2 / c2, -s3 / c2))


def _spec(j):
    return pl.BlockSpec((_L, _BC), lambda b, jj=j: (0, b + jj * _GRID))


_call = pl.pallas_call(
    _body,
    grid=(_GRID,),
    in_specs=[_spec(j) for _ in range(6) for j in range(_NSPLIT)],
    out_specs=pl.BlockSpec((1, 128), lambda b: (0, 0)),
    out_shape=jax.ShapeDtypeStruct((1, 128), jnp.float32),
    scratch_shapes=[pltpu.VMEM((5, _BC), jnp.float32)],
    compiler_params=pltpu.CompilerParams(
        dimension_semantics=("arbitrary",),
    ),
)


def kernel(seq, bn_seq, fg_seq, seqLogprobs, bnLogprobs, fgLogprobs, reward):
    del bn_seq
    arrs = (seq.T.astype(jnp.int32), fg_seq.T.astype(jnp.int32),
            seqLogprobs.T, bnLogprobs.T, fgLogprobs.T, reward.T)
    out = _call(*[a for a in arrs for _ in range(_NSPLIT)])
    return (out[0, 0], out[0, 1], out[0, 2])
